# baseline (device time: 6582 ns/iter reference)
import jax
import jax.numpy as jnp
from jax import lax
from jax.experimental import pallas as pl
from jax.experimental.pallas import tpu as pltpu

EPS = 1e-5
Y_SIZE = 2


def kernel(x, gamma):
    m, n = x.shape
    n_global = Y_SIZE * n
    blocks = m // 128

    def body(x_hbm, g_hbm, out_hbm,
             xv_ref, g_ref, out_vmem, partial_ref, recv_ref,
             load_sems, out_sem, send_sem, recv_sem):
        my_x = lax.axis_index("x")
        my_y = lax.axis_index("y")
        nbr = (my_x, 1 - my_y)

        cp_x = pltpu.make_async_copy(x_hbm, xv_ref, load_sems.at[0])
        cp_x.start()
        cp_g = pltpu.make_async_copy(g_hbm, g_ref, load_sems.at[1])
        cp_g.start()

        barrier_sem = pltpu.get_barrier_semaphore()
        pl.semaphore_signal(
            barrier_sem, inc=1, device_id=nbr,
            device_id_type=pl.DeviceIdType.MESH,
        )

        cp_x.wait()
        xv = xv_ref[:, :]
        x3 = xv.reshape(blocks, 128, n)
        partial_ref[:, :] = jnp.sum(x3 * x3, axis=2)

        cp_g.wait()
        out_vmem[:, :] = g_ref[:, :] * xv

        pl.semaphore_wait(barrier_sem, 1)

        rdma = pltpu.make_async_remote_copy(
            src_ref=partial_ref,
            dst_ref=recv_ref,
            send_sem=send_sem,
            recv_sem=recv_sem,
            device_id=nbr,
            device_id_type=pl.DeviceIdType.MESH,
        )
        rdma.start()
        rdma.wait()

        total = partial_ref[:, :] + recv_ref[:, :]
        inv = lax.rsqrt(total / n_global + EPS)
        out3 = out_vmem[:, :].reshape(blocks, 128, n) * inv[:, :, None]
        out_vmem[:, :] = out3.reshape(m, n)

        cp_out = pltpu.make_async_copy(out_vmem, out_hbm, out_sem)
        cp_out.start()
        cp_out.wait()

    return pl.pallas_call(
        body,
        out_shape=jax.ShapeDtypeStruct((m, n), jnp.float32),
        in_specs=[
            pl.BlockSpec(memory_space=pl.ANY),
            pl.BlockSpec(memory_space=pl.ANY),
        ],
        out_specs=pl.BlockSpec(memory_space=pl.ANY),
        scratch_shapes=[
            pltpu.VMEM((m, n), jnp.float32),
            pltpu.VMEM((1, n), jnp.float32),
            pltpu.VMEM((m, n), jnp.float32),
            pltpu.VMEM((m // 128, 128), jnp.float32),
            pltpu.VMEM((m // 128, 128), jnp.float32),
            pltpu.SemaphoreType.DMA((2,)),
            pltpu.SemaphoreType.DMA,
            pltpu.SemaphoreType.DMA,
            pltpu.SemaphoreType.DMA,
        ],
        compiler_params=pltpu.CompilerParams(collective_id=0),
        input_output_aliases={0: 0},
    )(x, gamma.reshape(1, n))


# device time: 5834 ns/iter; 1.1282x vs baseline; 1.1282x over previous
import jax
import jax.numpy as jnp
from jax import lax
from jax.experimental import pallas as pl
from jax.experimental.pallas import tpu as pltpu

EPS = 1e-5
Y_SIZE = 2


def kernel(x, gamma):
    m, n = x.shape
    n_global = Y_SIZE * n
    blocks = m // 128

    def body(x_hbm, g_hbm, out_hbm,
             xv_ref, g_ref, out_vmem, partial_ref, recv_ref,
             load_sems, out_sem, send_sem, recv_sem):
        my_x = lax.axis_index("x")
        my_y = lax.axis_index("y")
        nbr = (my_x, 1 - my_y)

        cp_x = pltpu.make_async_copy(x_hbm, xv_ref, load_sems.at[0])
        cp_x.start()
        cp_g = pltpu.make_async_copy(g_hbm, g_ref, load_sems.at[1])
        cp_g.start()

        barrier_sem = pltpu.get_barrier_semaphore()
        pl.semaphore_signal(
            barrier_sem, inc=1, device_id=nbr,
            device_id_type=pl.DeviceIdType.MESH,
        )

        cp_x.wait()
        xv = xv_ref[:, :]
        x3 = xv.reshape(blocks, 128, n)
        partial_ref[:, :] = jnp.sum(x3 * x3, axis=2)

        cp_g.wait()
        out_vmem[:, :] = g_ref[:, :] * xv

        pl.semaphore_wait(barrier_sem, 1)

        rdma = pltpu.make_async_remote_copy(
            src_ref=partial_ref,
            dst_ref=recv_ref,
            send_sem=send_sem,
            recv_sem=recv_sem,
            device_id=nbr,
            device_id_type=pl.DeviceIdType.MESH,
        )
        rdma.start()
        rdma.wait()

        total = partial_ref[:, :] + recv_ref[:, :]
        inv = lax.rsqrt(total / n_global + EPS)
        out3 = out_vmem[:, :].reshape(blocks, 128, n) * inv[:, :, None]
        out_vmem[:, :] = out3.reshape(m, n)

        cp_out = pltpu.make_async_copy(out_vmem, out_hbm, out_sem)
        cp_out.start()
        cp_out.wait()

    return pl.pallas_call(
        body,
        out_shape=jax.ShapeDtypeStruct((m, n), jnp.float32),
        in_specs=[
            pl.BlockSpec(memory_space=pl.ANY),
            pl.BlockSpec(memory_space=pl.ANY),
        ],
        out_specs=pl.BlockSpec(memory_space=pl.ANY),
        scratch_shapes=[
            pltpu.VMEM((m, n), jnp.float32),
            pltpu.VMEM((1, n), jnp.float32),
            pltpu.VMEM((m, n), jnp.float32),
            pltpu.VMEM((m // 128, 128), jnp.float32),
            pltpu.VMEM((m // 128, 128), jnp.float32),
            pltpu.SemaphoreType.DMA((2,)),
            pltpu.SemaphoreType.DMA,
            pltpu.SemaphoreType.DMA,
            pltpu.SemaphoreType.DMA,
        ],
        compiler_params=pltpu.CompilerParams(collective_id=0),
    )(
        pltpu.with_memory_space_constraint(x, pltpu.MemorySpace.HBM),
        pltpu.with_memory_space_constraint(
            gamma.reshape(1, n), pltpu.MemorySpace.HBM
        ),
    )
